# baseline (speedup 1.0000x reference)
import functools

import jax
import jax.numpy as jnp
from jax import lax
from jax.experimental import pallas as pl
from jax.experimental.pallas import tpu as pltpu
from jax.experimental.pallas import tpu_sc as plsc

CUTOFF = 0.5
E_PARENT = 6400000
E_FILTER = 1600000

NW = 32
NBLK = E_FILTER // 128
BASE_BLKS = NBLK // NW
EXTRA = NBLK - BASE_BLKS * NW

CB_A = 78
CN_A = CB_A * 128
NCH_A = BASE_BLKS // CB_A

CB_B = 39
CN_B = CB_B * 128
NCH_B = BASE_BLKS // CB_B

_PI = 3.14159265358979
_C3 = -1.0 / 6.0
_C5 = 1.0 / 120.0
_C7 = -1.0 / 5040.0
_C9 = 1.0 / 362880.0


def _expand_step(idx_v, idxf_v, g):
    a = idx_v[pl.ds(g * 16, 16)]
    p0 = (a >> 7) * 384 + (a & 127)
    b = g >> 3
    r = g & 7
    dst = b * 384 + r * 16
    idxf_v[pl.ds(dst, 16)] = p0
    idxf_v[pl.ds(dst + 128, 16)] = p0 + 128
    idxf_v[pl.ds(dst + 256, 16)] = p0 + 256


def _switch_step(d_v, sw_v, m_v, i):
    d16 = d_v[pl.ds(i * 16, 16)]
    s = (d16 * (2.0 * _PI)) - (0.5 * _PI)
    s2 = s * s
    p = s2 * _C9 + _C7
    p = s2 * p + _C5
    p = s2 * p + _C3
    sin_s = s + s * (s2 * p)
    mask = d16 < CUTOFF
    sw_v[pl.ds(i * 16, 16)] = jnp.where(mask, 0.5 - 0.5 * sin_s,
                                        jnp.zeros((16,), jnp.float32))
    m_v[pl.ds(i * 16, 16)] = jnp.where(mask, jnp.ones((16,), jnp.int32),
                                       jnp.zeros((16,), jnp.int32))


def _worker_start(_=None):
    wid = lax.axis_index("s") * 2 + lax.axis_index("c")
    return wid, wid * BASE_BLKS + jnp.minimum(wid, EXTRA)


@functools.partial(
    pl.kernel,
    out_type=jax.ShapeDtypeStruct((E_FILTER,), jnp.float32),
    mesh=plsc.VectorSubcoreMesh(core_axis_name="c", subcore_axis_name="s"),
    scratch_types=[
        pltpu.VMEM((CN_A,), jnp.int32),
        pltpu.VMEM((CN_A,), jnp.int32),
        pltpu.VMEM((CN_A,), jnp.float32),
        pltpu.VMEM((CN_A,), jnp.float32),
        pltpu.SemaphoreType.DMA,
        pltpu.SemaphoreType.DMA,
        pltpu.SemaphoreType.DMA,
        pltpu.SemaphoreType.DMA,
    ],
)
def _dist_gather(dist_hbm, idx_hbm, d_out,
                 idx0, idx1, d0, d1, sem0, sem1, osem0, osem1):
    wid, start_blk = _worker_start()
    idxs = (idx0, idx1)
    ds = (d0, d1)
    sems = (sem0, sem1)
    osems = (osem0, osem1)

    def stage_fire(j):
        p = j % 2
        base_i = (start_blk + j * CB_A) * 128
        pltpu.sync_copy(idx_hbm.at[pl.ds(base_i, CN_A)], idxs[p])
        return pltpu.async_copy(dist_hbm.at[idxs[p]], ds[p], sems[p])

    cps = [stage_fire(0), None]
    outs = [None, None]
    for j in range(NCH_A):
        p = j % 2
        q = (j + 1) % 2
        if j + 1 < NCH_A:
            if outs[q] is not None:
                outs[q].wait()
                outs[q] = None
            cps[q] = stage_fire(j + 1)
        cps[p].wait()
        if outs[p] is not None:
            outs[p].wait()
            outs[p] = None
        base_i = (start_blk + j * CB_A) * 128
        outs[p] = pltpu.async_copy(ds[p], d_out.at[pl.ds(base_i, CN_A)],
                                   osems[p])

    for h in outs:
        if h is not None:
            h.wait()

    @pl.when(wid < EXTRA)
    def _():
        base_i = (start_blk + BASE_BLKS) * 128
        pltpu.sync_copy(idx_hbm.at[pl.ds(base_i, 128)], idx0.at[pl.ds(0, 128)])
        pltpu.async_copy(dist_hbm.at[idx0.at[pl.ds(0, 128)]],
                         d0.at[pl.ds(0, 128)], sem0).wait()
        pltpu.sync_copy(d0.at[pl.ds(0, 128)], d_out.at[pl.ds(base_i, 128)])


def _switch_tc_body(d_ref, sw_ref, m_ref):
    d = d_ref[...]
    mask = d < CUTOFF
    sw_ref[...] = jnp.where(mask, 0.5 * jnp.cos((jnp.pi / CUTOFF) * d) + 0.5,
                            0.0)
    m_ref[...] = mask.astype(jnp.int32)


_switch_tc = pl.pallas_call(
    _switch_tc_body,
    out_shape=[
        jax.ShapeDtypeStruct((E_FILTER,), jnp.float32),
        jax.ShapeDtypeStruct((E_FILTER,), jnp.int32),
    ],
)


@functools.partial(
    pl.kernel,
    out_type=[
        jax.ShapeDtypeStruct((3 * E_FILTER,), jnp.float32),
        jax.ShapeDtypeStruct((E_FILTER,), jnp.float32),
    ],
    mesh=plsc.VectorSubcoreMesh(core_axis_name="c", subcore_axis_name="s"),
    scratch_types=[
        pltpu.VMEM((CN_B,), jnp.int32),
        pltpu.VMEM((CN_B,), jnp.int32),
        pltpu.VMEM((3 * CN_B,), jnp.int32),
        pltpu.VMEM((3 * CN_B,), jnp.int32),
        pltpu.VMEM((3 * CN_B,), jnp.float32),
        pltpu.VMEM((3 * CN_B,), jnp.float32),
        pltpu.VMEM((CN_B,), jnp.float32),
        pltpu.VMEM((CN_B,), jnp.float32),
        pltpu.SemaphoreType.DMA,
        pltpu.SemaphoreType.DMA,
        pltpu.SemaphoreType.DMA,
        pltpu.SemaphoreType.DMA,
    ],
)
def _vec_gather(vpf_hbm, dist_hbm, idx_hbm, v_out, d_out,
                idx0, idx1, idxf0, idxf1, v0, v1, d0, d1,
                sem0, sem1, osem0, osem1):
    wid, start_blk = _worker_start()
    idxs = (idx0, idx1)
    idxfs = (idxf0, idxf1)
    vs = (v0, v1)
    dbs = (d0, d1)
    sems = (sem0, sem1)
    osems = (osem0, osem1)

    def stage_expand(j, idxf_v):
        p = j % 2
        base_i = (start_blk + j * CB_B) * 128
        pltpu.sync_copy(idx_hbm.at[pl.ds(base_i, CN_B)], idxs[p])

        def expand(g, carry2):
            _expand_step(idxs[p], idxf_v, g)
            return carry2

        lax.fori_loop(0, CN_B // 16, expand, 0, unroll=4)

    def fire(q):
        return (pltpu.async_copy(vpf_hbm.at[idxfs[q]], vs[q], sems[q]),
                pltpu.async_copy(dist_hbm.at[idxs[q]], dbs[q], sems[q]))

    stage_expand(0, idxf0)
    cps = [fire(0), None]
    outs = [None, None]
    for j in range(NCH_B):
        p = j % 2
        q = (j + 1) % 2
        if j + 1 < NCH_B:
            stage_expand(j + 1, idxfs[q])
            if outs[q] is not None:
                for h in outs[q]:
                    h.wait()
                outs[q] = None
            cps[q] = fire(q)
        else:
            if outs[q] is not None:
                for h in outs[q]:
                    h.wait()
                outs[q] = None

            @pl.when(wid < EXTRA)
            def _():
                base_i = (start_blk + BASE_BLKS) * 128
                pltpu.sync_copy(idx_hbm.at[pl.ds(base_i, 128)],
                                idxs[q].at[pl.ds(0, 128)])

                def expand2(g, carry2):
                    _expand_step(idxs[q], idxfs[q], g)
                    return carry2

                lax.fori_loop(0, 8, expand2, 0, unroll=4)
                cpv = pltpu.async_copy(vpf_hbm.at[idxfs[q].at[pl.ds(0, 384)]],
                                       vs[q].at[pl.ds(0, 384)], sems[q])
                cpd = pltpu.async_copy(dist_hbm.at[idxs[q].at[pl.ds(0, 128)]],
                                       dbs[q].at[pl.ds(0, 128)], sems[q])
                cpv.wait()
                cpd.wait()
                pltpu.sync_copy(vs[q].at[pl.ds(0, 384)],
                                v_out.at[pl.ds(base_i * 3, 384)])
                pltpu.sync_copy(dbs[q].at[pl.ds(0, 128)],
                                d_out.at[pl.ds(base_i, 128)])
        base_i = (start_blk + j * CB_B) * 128
        for h in cps[p]:
            h.wait()
        outs[p] = (
            pltpu.async_copy(vs[p], v_out.at[pl.ds(base_i * 3, 3 * CN_B)],
                             osems[p]),
            pltpu.async_copy(dbs[p], d_out.at[pl.ds(base_i, CN_B)], osems[p]),
        )

    for pair in outs:
        if pair is not None:
            for h in pair:
                h.wait()


def kernel(vec, distances, filter_indices):
    vpf = vec.T.reshape(3, E_PARENT // 128, 128).transpose(1, 0, 2).reshape(-1)
    vflat, d = _vec_gather(vpf, distances, filter_indices)
    sw, m = _switch_tc(d)
    v = vflat.reshape(E_FILTER // 128, 3, 128).transpose(1, 0, 2)
    v = v.reshape(3, E_FILTER).T
    return v, d, sw, m.astype(jnp.bool_)


# speedup vs baseline: 1.1637x; 1.1637x over previous
import functools

import jax
import jax.numpy as jnp
from jax import lax
from jax.experimental import pallas as pl
from jax.experimental.pallas import tpu as pltpu
from jax.experimental.pallas import tpu_sc as plsc

CUTOFF = 0.5
E_PARENT = 6400000
E_FILTER = 1600000

NW = 32
NBLK = E_FILTER // 128
BASE_BLKS = NBLK // NW
EXTRA = NBLK - BASE_BLKS * NW

CB_A = 78
CN_A = CB_A * 128
NCH_A = BASE_BLKS // CB_A

CB_B = 65
CN_B = CB_B * 128
NCH_B = BASE_BLKS // CB_B

_PI = 3.14159265358979
_C3 = -1.0 / 6.0
_C5 = 1.0 / 120.0
_C7 = -1.0 / 5040.0
_C9 = 1.0 / 362880.0


def _expand_step(idx_v, idxf_v, g):
    a = idx_v[pl.ds(g * 16, 16)]
    p0 = (a >> 7) * 384 + (a & 127)
    b = g >> 3
    r = g & 7
    dst = b * 384 + r * 16
    idxf_v[pl.ds(dst, 16)] = p0
    idxf_v[pl.ds(dst + 128, 16)] = p0 + 128
    idxf_v[pl.ds(dst + 256, 16)] = p0 + 256


def _switch_step(d_v, sw_v, m_v, i):
    d16 = d_v[pl.ds(i * 16, 16)]
    s = (d16 * (2.0 * _PI)) - (0.5 * _PI)
    s2 = s * s
    p = s2 * _C9 + _C7
    p = s2 * p + _C5
    p = s2 * p + _C3
    sin_s = s + s * (s2 * p)
    mask = d16 < CUTOFF
    sw_v[pl.ds(i * 16, 16)] = jnp.where(mask, 0.5 - 0.5 * sin_s,
                                        jnp.zeros((16,), jnp.float32))
    m_v[pl.ds(i * 16, 16)] = jnp.where(mask, jnp.ones((16,), jnp.int32),
                                       jnp.zeros((16,), jnp.int32))


def _worker_start(_=None):
    wid = lax.axis_index("s") * 2 + lax.axis_index("c")
    return wid, wid * BASE_BLKS + jnp.minimum(wid, EXTRA)


@functools.partial(
    pl.kernel,
    out_type=jax.ShapeDtypeStruct((E_FILTER,), jnp.float32),
    mesh=plsc.VectorSubcoreMesh(core_axis_name="c", subcore_axis_name="s"),
    scratch_types=[
        pltpu.VMEM((CN_A,), jnp.int32),
        pltpu.VMEM((CN_A,), jnp.int32),
        pltpu.VMEM((CN_A,), jnp.float32),
        pltpu.VMEM((CN_A,), jnp.float32),
        pltpu.SemaphoreType.DMA,
        pltpu.SemaphoreType.DMA,
        pltpu.SemaphoreType.DMA,
        pltpu.SemaphoreType.DMA,
    ],
)
def _dist_gather(dist_hbm, idx_hbm, d_out,
                 idx0, idx1, d0, d1, sem0, sem1, osem0, osem1):
    wid, start_blk = _worker_start()
    idxs = (idx0, idx1)
    ds = (d0, d1)
    sems = (sem0, sem1)
    osems = (osem0, osem1)

    def stage_fire(j):
        p = j % 2
        base_i = (start_blk + j * CB_A) * 128
        pltpu.sync_copy(idx_hbm.at[pl.ds(base_i, CN_A)], idxs[p])
        return pltpu.async_copy(dist_hbm.at[idxs[p]], ds[p], sems[p])

    cps = [stage_fire(0), None]
    outs = [None, None]
    for j in range(NCH_A):
        p = j % 2
        q = (j + 1) % 2
        if j + 1 < NCH_A:
            if outs[q] is not None:
                outs[q].wait()
                outs[q] = None
            cps[q] = stage_fire(j + 1)
        cps[p].wait()
        if outs[p] is not None:
            outs[p].wait()
            outs[p] = None
        base_i = (start_blk + j * CB_A) * 128
        outs[p] = pltpu.async_copy(ds[p], d_out.at[pl.ds(base_i, CN_A)],
                                   osems[p])

    for h in outs:
        if h is not None:
            h.wait()

    @pl.when(wid < EXTRA)
    def _():
        base_i = (start_blk + BASE_BLKS) * 128
        pltpu.sync_copy(idx_hbm.at[pl.ds(base_i, 128)], idx0.at[pl.ds(0, 128)])
        pltpu.async_copy(dist_hbm.at[idx0.at[pl.ds(0, 128)]],
                         d0.at[pl.ds(0, 128)], sem0).wait()
        pltpu.sync_copy(d0.at[pl.ds(0, 128)], d_out.at[pl.ds(base_i, 128)])


def _switch_tc_body(d_ref, sw_ref, m_ref):
    d = d_ref[...]
    mask = d < CUTOFF
    sw_ref[...] = jnp.where(mask, 0.5 * jnp.cos((jnp.pi / CUTOFF) * d) + 0.5,
                            0.0)
    m_ref[...] = mask.astype(jnp.int32)


_switch_tc = pl.pallas_call(
    _switch_tc_body,
    out_shape=[
        jax.ShapeDtypeStruct((E_FILTER,), jnp.float32),
        jax.ShapeDtypeStruct((E_FILTER,), jnp.int32),
    ],
)


@functools.partial(
    pl.kernel,
    out_type=jax.ShapeDtypeStruct((3 * E_FILTER,), jnp.float32),
    mesh=plsc.VectorSubcoreMesh(core_axis_name="c", subcore_axis_name="s"),
    scratch_types=[
        pltpu.VMEM((CN_B,), jnp.int32),
        pltpu.VMEM((3 * CN_B,), jnp.int32),
        pltpu.VMEM((3 * CN_B,), jnp.int32),
        pltpu.VMEM((3 * CN_B,), jnp.float32),
        pltpu.VMEM((3 * CN_B,), jnp.float32),
        pltpu.SemaphoreType.DMA,
        pltpu.SemaphoreType.DMA,
        pltpu.SemaphoreType.DMA,
        pltpu.SemaphoreType.DMA,
    ],
)
def _vec_gather(vpf_hbm, idx_hbm, v_out,
                idx_v, idxf0, idxf1, v0, v1, sem0, sem1, osem0, osem1):
    wid, start_blk = _worker_start()
    idxfs = (idxf0, idxf1)
    vs = (v0, v1)
    sems = (sem0, sem1)
    osems = (osem0, osem1)

    def stage_expand(j, idxf_v):
        base_i = (start_blk + j * CB_B) * 128
        pltpu.sync_copy(idx_hbm.at[pl.ds(base_i, CN_B)], idx_v)

        def expand(g, carry2):
            _expand_step(idx_v, idxf_v, g)
            return carry2

        lax.fori_loop(0, CN_B // 16, expand, 0, unroll=4)

    stage_expand(0, idxf0)
    cps = [pltpu.async_copy(vpf_hbm.at[idxf0], v0, sem0), None]
    outs = [None, None]
    for j in range(NCH_B):
        p = j % 2
        q = (j + 1) % 2
        if j + 1 < NCH_B:
            stage_expand(j + 1, idxfs[q])
            if outs[q] is not None:
                outs[q].wait()
                outs[q] = None
            cps[q] = pltpu.async_copy(vpf_hbm.at[idxfs[q]], vs[q], sems[q])
        else:
            if outs[q] is not None:
                outs[q].wait()
                outs[q] = None

            @pl.when(wid < EXTRA)
            def _():
                base_i = (start_blk + BASE_BLKS) * 128
                pltpu.sync_copy(idx_hbm.at[pl.ds(base_i, 128)],
                                idx_v.at[pl.ds(0, 128)])

                def expand2(g, carry2):
                    _expand_step(idx_v, idxfs[q], g)
                    return carry2

                lax.fori_loop(0, 8, expand2, 0, unroll=4)
                pltpu.async_copy(vpf_hbm.at[idxfs[q].at[pl.ds(0, 384)]],
                                 vs[q].at[pl.ds(0, 384)], sems[q]).wait()
                pltpu.sync_copy(vs[q].at[pl.ds(0, 384)],
                                v_out.at[pl.ds(base_i * 3, 384)])
        base_i = (start_blk + j * CB_B) * 128
        cps[p].wait()
        outs[p] = pltpu.async_copy(vs[p], v_out.at[pl.ds(base_i * 3, 3 * CN_B)],
                                   osems[p])

    for h in outs:
        if h is not None:
            h.wait()


def kernel(vec, distances, filter_indices):
    vpf = vec.T.reshape(3, E_PARENT // 128, 128).transpose(1, 0, 2).reshape(-1)
    d = _dist_gather(distances, filter_indices)
    vflat = _vec_gather(vpf, filter_indices)
    sw, m = _switch_tc(d)
    v = vflat.reshape(E_FILTER // 128, 3, 128).transpose(1, 0, 2)
    v = v.reshape(3, E_FILTER).T
    return v, d, sw, m.astype(jnp.bool_)


# speedup vs baseline: 1.1638x; 1.0001x over previous
import functools

import jax
import jax.numpy as jnp
from jax import lax
from jax.experimental import pallas as pl
from jax.experimental.pallas import tpu as pltpu
from jax.experimental.pallas import tpu_sc as plsc

CUTOFF = 0.5
E_PARENT = 6400000
E_FILTER = 1600000

NW = 32
NBLK = E_FILTER // 128
BASE_BLKS = NBLK // NW
EXTRA = NBLK - BASE_BLKS * NW

CB_A = 78
CN_A = CB_A * 128
NCH_A = BASE_BLKS // CB_A

CB_B = 65
CN_B = CB_B * 128
NCH_B = BASE_BLKS // CB_B

def _expand_step(idx_v, idxf_v, g):
    a = idx_v[pl.ds(g * 16, 16)]
    p0 = (a >> 7) * 384 + (a & 127)
    b = g >> 3
    r = g & 7
    dst = b * 384 + r * 16
    idxf_v[pl.ds(dst, 16)] = p0
    idxf_v[pl.ds(dst + 128, 16)] = p0 + 128
    idxf_v[pl.ds(dst + 256, 16)] = p0 + 256


def _worker_start(_=None):
    wid = lax.axis_index("s") * 2 + lax.axis_index("c")
    return wid, wid * BASE_BLKS + jnp.minimum(wid, EXTRA)


@functools.partial(
    pl.kernel,
    out_type=jax.ShapeDtypeStruct((E_FILTER,), jnp.float32),
    mesh=plsc.VectorSubcoreMesh(core_axis_name="c", subcore_axis_name="s"),
    scratch_types=[
        pltpu.VMEM((CN_A,), jnp.int32),
        pltpu.VMEM((CN_A,), jnp.int32),
        pltpu.VMEM((CN_A,), jnp.float32),
        pltpu.VMEM((CN_A,), jnp.float32),
        pltpu.SemaphoreType.DMA,
        pltpu.SemaphoreType.DMA,
        pltpu.SemaphoreType.DMA,
        pltpu.SemaphoreType.DMA,
    ],
)
def _dist_gather(dist_hbm, idx_hbm, d_out,
                 idx0, idx1, d0, d1, sem0, sem1, osem0, osem1):
    wid, start_blk = _worker_start()
    idxs = (idx0, idx1)
    ds = (d0, d1)
    sems = (sem0, sem1)
    osems = (osem0, osem1)

    def stage_fire(j):
        p = j % 2
        base_i = (start_blk + j * CB_A) * 128
        pltpu.sync_copy(idx_hbm.at[pl.ds(base_i, CN_A)], idxs[p])
        return pltpu.async_copy(dist_hbm.at[idxs[p]], ds[p], sems[p])

    cps = [stage_fire(0), None]
    outs = [None, None]
    for j in range(NCH_A):
        p = j % 2
        q = (j + 1) % 2
        if j + 1 < NCH_A:
            if outs[q] is not None:
                outs[q].wait()
                outs[q] = None
            cps[q] = stage_fire(j + 1)
        cps[p].wait()
        if outs[p] is not None:
            outs[p].wait()
            outs[p] = None
        base_i = (start_blk + j * CB_A) * 128
        outs[p] = pltpu.async_copy(ds[p], d_out.at[pl.ds(base_i, CN_A)],
                                   osems[p])

    for h in outs:
        if h is not None:
            h.wait()

    @pl.when(wid < EXTRA)
    def _():
        base_i = (start_blk + BASE_BLKS) * 128
        pltpu.sync_copy(idx_hbm.at[pl.ds(base_i, 128)], idx0.at[pl.ds(0, 128)])
        pltpu.async_copy(dist_hbm.at[idx0.at[pl.ds(0, 128)]],
                         d0.at[pl.ds(0, 128)], sem0).wait()
        pltpu.sync_copy(d0.at[pl.ds(0, 128)], d_out.at[pl.ds(base_i, 128)])


def _switch_tc_body(d_ref, sw_ref, m_ref):
    d = d_ref[...]
    mask = d < CUTOFF
    sw_ref[...] = jnp.where(mask, 0.5 * jnp.cos((jnp.pi / CUTOFF) * d) + 0.5,
                            0.0)
    m_ref[...] = mask.astype(jnp.int32)


_switch_tc = pl.pallas_call(
    _switch_tc_body,
    out_shape=[
        jax.ShapeDtypeStruct((E_FILTER,), jnp.float32),
        jax.ShapeDtypeStruct((E_FILTER,), jnp.int32),
    ],
)


@functools.partial(
    pl.kernel,
    out_type=jax.ShapeDtypeStruct((3 * E_FILTER,), jnp.float32),
    mesh=plsc.VectorSubcoreMesh(core_axis_name="c", subcore_axis_name="s"),
    scratch_types=[
        pltpu.VMEM((CN_B,), jnp.int32),
        pltpu.VMEM((3 * CN_B,), jnp.int32),
        pltpu.VMEM((3 * CN_B,), jnp.int32),
        pltpu.VMEM((3 * CN_B,), jnp.float32),
        pltpu.VMEM((3 * CN_B,), jnp.float32),
        pltpu.SemaphoreType.DMA,
        pltpu.SemaphoreType.DMA,
        pltpu.SemaphoreType.DMA,
        pltpu.SemaphoreType.DMA,
    ],
)
def _vec_gather(vpf_hbm, idx_hbm, v_out,
                idx_v, idxf0, idxf1, v0, v1, sem0, sem1, osem0, osem1):
    wid, start_blk = _worker_start()
    idxfs = (idxf0, idxf1)
    vs = (v0, v1)
    sems = (sem0, sem1)
    osems = (osem0, osem1)

    def stage_expand(j, idxf_v):
        base_i = (start_blk + j * CB_B) * 128
        pltpu.sync_copy(idx_hbm.at[pl.ds(base_i, CN_B)], idx_v)

        def expand(g, carry2):
            _expand_step(idx_v, idxf_v, g)
            return carry2

        lax.fori_loop(0, CN_B // 16, expand, 0, unroll=4)

    stage_expand(0, idxf0)
    cps = [pltpu.async_copy(vpf_hbm.at[idxf0], v0, sem0), None]
    outs = [None, None]
    for j in range(NCH_B):
        p = j % 2
        q = (j + 1) % 2
        if j + 1 < NCH_B:
            stage_expand(j + 1, idxfs[q])
            if outs[q] is not None:
                outs[q].wait()
                outs[q] = None
            cps[q] = pltpu.async_copy(vpf_hbm.at[idxfs[q]], vs[q], sems[q])
        else:
            if outs[q] is not None:
                outs[q].wait()
                outs[q] = None

            @pl.when(wid < EXTRA)
            def _():
                base_i = (start_blk + BASE_BLKS) * 128
                pltpu.sync_copy(idx_hbm.at[pl.ds(base_i, 128)],
                                idx_v.at[pl.ds(0, 128)])

                def expand2(g, carry2):
                    _expand_step(idx_v, idxfs[q], g)
                    return carry2

                lax.fori_loop(0, 8, expand2, 0, unroll=4)
                pltpu.async_copy(vpf_hbm.at[idxfs[q].at[pl.ds(0, 384)]],
                                 vs[q].at[pl.ds(0, 384)], sems[q]).wait()
                pltpu.sync_copy(vs[q].at[pl.ds(0, 384)],
                                v_out.at[pl.ds(base_i * 3, 384)])
        base_i = (start_blk + j * CB_B) * 128
        cps[p].wait()
        outs[p] = pltpu.async_copy(vs[p], v_out.at[pl.ds(base_i * 3, 3 * CN_B)],
                                   osems[p])

    for h in outs:
        if h is not None:
            h.wait()


def kernel(vec, distances, filter_indices):
    vpf = vec.T.reshape(3, E_PARENT // 128, 128).transpose(1, 0, 2).reshape(-1)
    d = _dist_gather(distances, filter_indices)
    vflat = _vec_gather(vpf, filter_indices)
    sw, m = _switch_tc(d)
    v = vflat.reshape(E_FILTER // 128, 3, 128).transpose(1, 0, 2)
    v = v.reshape(3, E_FILTER).T
    return v, d, sw, m.astype(jnp.bool_)
